# Initial kernel scaffold; baseline (speedup 1.0000x reference)
import jax
import jax.numpy as jnp
from jax.experimental import pallas as pl


def kernel(pix_to_face, bary_coords, face_textures):
    raise NotImplementedError("write your pallas kernel here")


# speedup vs baseline: 3.1456x; 3.1456x over previous
import functools

import jax
import jax.numpy as jnp
from jax import lax
from jax.experimental import pallas as pl
from jax.experimental.pallas import tpu as pltpu
from jax.experimental.pallas import tpu_sc as plsc

_N, _H, _W, _K, _C = 8, 512, 512, 4, 3
_P = _N * _H * _W
_NW = 32
_PPW = _P // _NW
_B = 2048
_NCHUNK = _PPW // _B
_GROW = 128
_NG = _B // _GROW
_GPR = _GROW // 16
_D = 16


def _shader_body(pix_hbm, bary_hbm, tex_hbm, out_hbm,
                 pix_v, bary_v, idx_v, rows_v, out_v, sem):
    info = plsc.get_sparse_core_info()
    wid = lax.axis_index("s") * info.num_cores + lax.axis_index("c")
    base = wid * _PPW

    lane = lax.iota(jnp.int32, 16)
    cols_t = [jnp.full((16,), 3 * i + c, jnp.int32)
              for i in range(3) for c in range(3)]

    def chunk(g, _):
        off = base + g * _B
        pltpu.sync_copy(pix_hbm.at[pl.ds(off * _K, _B * _K)], pix_v)
        pltpu.sync_copy(bary_hbm.at[pl.ds(off * 12, _B * 12)], bary_v)

        copies = []
        for j in range(_NG):
            def mkidx(i, _, j=j):
                r = (j * _GPR + i) * 16 + lane
                f = plsc.load_gather(pix_v, [r * _K])
                idx_v[j, pl.ds(i * 16, 16)] = jnp.maximum(f, 0)
                return _
            lax.fori_loop(0, _GPR, mkidx, None)
            copies.append(pltpu.async_copy(
                tex_hbm.at[idx_v.at[j]],
                rows_v.at[pl.ds(j * _GROW, _GROW)], sem))
        for cp in copies:
            cp.wait()

        def comp(i, _):
            r = i * 16 + lane
            f = plsc.load_gather(pix_v, [r * _K])
            valid = f >= 0
            w = [plsc.load_gather(bary_v, [r * 12 + j]) for j in range(3)]
            for c in range(3):
                acc = w[0] * plsc.load_gather(rows_v, [r, cols_t[c]])
                acc += w[1] * plsc.load_gather(rows_v, [r, cols_t[3 + c]])
                acc += w[2] * plsc.load_gather(rows_v, [r, cols_t[6 + c]])
                acc = jnp.where(valid, acc, jnp.zeros((16,), jnp.float32))
                plsc.store_scatter(out_v, [r * 3 + c], acc)
            return _
        lax.fori_loop(0, _B // 16, comp, None)

        pltpu.sync_copy(out_v, out_hbm.at[pl.ds(off * 3, _B * 3)])
        return _
    lax.fori_loop(0, _NCHUNK, chunk, None)


def kernel(pix_to_face, bary_coords, face_textures):
    pix = pix_to_face.astype(jnp.int32).reshape(_P * _K)
    bary = bary_coords.reshape(_P * _K * 3)
    tex = face_textures.reshape(-1, 9)
    tex = jnp.pad(tex, ((0, 0), (0, _D - 9)))

    mesh = plsc.VectorSubcoreMesh(core_axis_name="c", subcore_axis_name="s")
    run = functools.partial(
        pl.kernel,
        mesh=mesh,
        compiler_params=pltpu.CompilerParams(
            needs_layout_passes=False, use_tc_tiling_on_sc=False),
        out_type=jax.ShapeDtypeStruct((_P * 3,), jnp.float32),
        scratch_types=[
            pltpu.VMEM((_B * _K,), jnp.int32),
            pltpu.VMEM((_B * 12,), jnp.float32),
            pltpu.VMEM((_NG, _GROW), jnp.int32),
            pltpu.VMEM((_B, _D), jnp.float32),
            pltpu.VMEM((_B * 3,), jnp.float32),
            pltpu.SemaphoreType.DMA,
        ],
    )(_shader_body)
    out = run(pix, bary, tex)
    return out.reshape(_N, _H, _W, _C)


# speedup vs baseline: 12.3021x; 3.9109x over previous
import functools

import jax
import jax.numpy as jnp
from jax import lax
from jax.experimental import pallas as pl
from jax.experimental.pallas import tpu as pltpu
from jax.experimental.pallas import tpu_sc as plsc

_N, _H, _W, _K, _C = 8, 512, 512, 4, 3
_P = _N * _H * _W
_NW = 32
_PPW = _P // _NW
_B = 2048
_NCHUNK = _PPW // _B
_GROW = 128
_NG = _B // _GROW
_GPR = _GROW // 16
_D = 16


def _shader_body(pix_hbm, bary_hbm, tex_hbm, out_hbm,
                 pix_v, bary_v, idx_v, rows_v, out_v, sem):
    info = plsc.get_sparse_core_info()
    wid = lax.axis_index("s") * info.num_cores + lax.axis_index("c")
    base = wid * _PPW

    lane = lax.iota(jnp.int32, 16)
    cols_t = [jnp.full((16,), 3 * i + c, jnp.int32)
              for i in range(3) for c in range(3)]

    def chunk(g, _):
        off = base + g * _B
        pltpu.sync_copy(pix_hbm.at[pl.ds(off, _B)], pix_v)
        pltpu.sync_copy(bary_hbm.at[pl.ds(off * 3, _B * 3)], bary_v)

        copies = []
        for j in range(_NG):
            def mkidx(i, _, j=j):
                f = pix_v[pl.ds((j * _GPR + i) * 16, 16)]
                idx_v[j, pl.ds(i * 16, 16)] = jnp.maximum(f, 0)
                return _
            lax.fori_loop(0, _GPR, mkidx, None)
            copies.append(pltpu.async_copy(
                tex_hbm.at[idx_v.at[j]],
                rows_v.at[pl.ds(j * _GROW, _GROW)], sem))
        for cp in copies:
            cp.wait()

        def comp(i, _):
            r = i * 16 + lane
            f = pix_v[pl.ds(i * 16, 16)]
            valid = f >= 0
            w = [plsc.load_gather(bary_v, [r * 3 + j]) for j in range(3)]
            for c in range(3):
                acc = w[0] * plsc.load_gather(rows_v, [r, cols_t[c]])
                acc += w[1] * plsc.load_gather(rows_v, [r, cols_t[3 + c]])
                acc += w[2] * plsc.load_gather(rows_v, [r, cols_t[6 + c]])
                acc = jnp.where(valid, acc, jnp.zeros((16,), jnp.float32))
                plsc.store_scatter(out_v, [r * 3 + c], acc)
            return _
        lax.fori_loop(0, _B // 16, comp, None)

        pltpu.sync_copy(out_v, out_hbm.at[pl.ds(off * 3, _B * 3)])
        return _
    lax.fori_loop(0, _NCHUNK, chunk, None)


def kernel(pix_to_face, bary_coords, face_textures):
    pix = pix_to_face[..., 0].astype(jnp.int32).reshape(_P)
    bary = bary_coords[:, :, :, 0, :].reshape(_P * 3)
    tex = face_textures.reshape(-1, 9)
    tex = jnp.pad(tex, ((0, 0), (0, _D - 9)))

    mesh = plsc.VectorSubcoreMesh(core_axis_name="c", subcore_axis_name="s")
    run = functools.partial(
        pl.kernel,
        mesh=mesh,
        compiler_params=pltpu.CompilerParams(
            needs_layout_passes=False, use_tc_tiling_on_sc=False),
        out_type=jax.ShapeDtypeStruct((_P * 3,), jnp.float32),
        scratch_types=[
            pltpu.VMEM((_B,), jnp.int32),
            pltpu.VMEM((_B * 3,), jnp.float32),
            pltpu.VMEM((_NG, _GROW), jnp.int32),
            pltpu.VMEM((_B, _D), jnp.float32),
            pltpu.VMEM((_B * 3,), jnp.float32),
            pltpu.SemaphoreType.DMA,
        ],
    )(_shader_body)
    out = run(pix, bary, tex)
    return out.reshape(_N, _H, _W, _C)


# speedup vs baseline: 56.9994x; 4.6333x over previous
import functools

import jax
import jax.numpy as jnp
from jax import lax
from jax.experimental import pallas as pl
from jax.experimental.pallas import tpu as pltpu
from jax.experimental.pallas import tpu_sc as plsc

_N, _H, _W, _K, _C = 8, 512, 512, 4, 3
_P = _N * _H * _W
_NH = _N * _H
_NW = 32
_PPW = _P // _NW
_B = 2048
_RPC = _B // _W
_NCHUNK = _PPW // _B
_GROW = 128
_NG = _B // _GROW
_GPR = _GROW // 16
_D = 16


def _shader_body(pix_hbm, bary_hbm, tex_hbm, out_hbm,
                 pix_v, bary_v, idx_v, rows_v, out_v, sem):
    info = plsc.get_sparse_core_info()
    wid = lax.axis_index("s") * info.num_cores + lax.axis_index("c")
    base = wid * (_PPW // _W)

    lane = lax.iota(jnp.int32, 16)
    cols_t = [jnp.full((16,), 3 * i + c, jnp.int32)
              for i in range(3) for c in range(3)]

    def chunk(g, _):
        nh0 = base + g * _RPC
        pltpu.sync_copy(pix_hbm.at[pl.ds(nh0, _RPC), 0], pix_v)
        for j in range(3):
            pltpu.sync_copy(bary_hbm.at[pl.ds(nh0, _RPC), j, 0],
                            bary_v.at[j])

        copies = []
        for j in range(_NG):
            def mkidx(i, _, j=j):
                ii = j * _GPR + i
                f = pix_v[ii // 32, pl.ds((ii % 32) * 16, 16)]
                idx_v[j, pl.ds(i * 16, 16)] = jnp.maximum(f, 0)
                return _
            lax.fori_loop(0, _GPR, mkidx, None)
            copies.append(pltpu.async_copy(
                tex_hbm.at[idx_v.at[j]],
                rows_v.at[pl.ds(j * _GROW, _GROW)], sem))
        for cp in copies:
            cp.wait()

        def comp(i, _):
            r = i * 16 + lane
            row, col = i // 32, (i % 32) * 16
            f = pix_v[row, pl.ds(col, 16)]
            valid = f >= 0
            w = [bary_v[j, row, pl.ds(col, 16)] for j in range(3)]
            for c in range(3):
                acc = w[0] * plsc.load_gather(rows_v, [r, cols_t[c]])
                acc += w[1] * plsc.load_gather(rows_v, [r, cols_t[3 + c]])
                acc += w[2] * plsc.load_gather(rows_v, [r, cols_t[6 + c]])
                acc = jnp.where(valid, acc, jnp.zeros((16,), jnp.float32))
                plsc.store_scatter(out_v, [r * 3 + c], acc)
            return _
        lax.fori_loop(0, _B // 16, comp, None)

        pltpu.sync_copy(out_v, out_hbm.at[pl.ds(nh0 * _W * 3, _B * 3)])
        return _
    lax.fori_loop(0, _NCHUNK, chunk, None)


def kernel(pix_to_face, bary_coords, face_textures):
    pix = jnp.transpose(pix_to_face.astype(jnp.int32),
                        (0, 1, 3, 2)).reshape(_NH, _K, _W)
    bary = jnp.transpose(bary_coords,
                         (0, 1, 4, 3, 2)).reshape(_NH, 3, _K, _W)
    tex = face_textures.reshape(-1, 9)
    tex = jnp.pad(tex, ((0, 0), (0, _D - 9)))

    mesh = plsc.VectorSubcoreMesh(core_axis_name="c", subcore_axis_name="s")
    run = functools.partial(
        pl.kernel,
        mesh=mesh,
        compiler_params=pltpu.CompilerParams(
            needs_layout_passes=False, use_tc_tiling_on_sc=False),
        out_type=jax.ShapeDtypeStruct((_P * 3,), jnp.float32),
        scratch_types=[
            pltpu.VMEM((_RPC, _W), jnp.int32),
            pltpu.VMEM((3, _RPC, _W), jnp.float32),
            pltpu.VMEM((_NG, _GROW), jnp.int32),
            pltpu.VMEM((_B, _D), jnp.float32),
            pltpu.VMEM((_B * 3,), jnp.float32),
            pltpu.SemaphoreType.DMA,
        ],
    )(_shader_body)
    out = run(pix, bary, tex)
    return out.reshape(_N, _H, _W, _C)


# speedup vs baseline: 59.5484x; 1.0447x over previous
import functools

import jax
import jax.numpy as jnp
from jax import lax
from jax.experimental import pallas as pl
from jax.experimental.pallas import tpu as pltpu
from jax.experimental.pallas import tpu_sc as plsc

_N, _H, _W, _K, _C = 8, 512, 512, 4, 3
_P = _N * _H * _W
_NH = _N * _H
_NW = 32
_PPW = _P // _NW
_B = 2048
_RPC = _B // _W
_NCHUNK = _PPW // _B
_GROW = 128
_NG = _B // _GROW
_GPR = _GROW // 16
_D = 16
_WT = _W // 128


def _shader_body(pix_hbm, bary_hbm, tex_hbm, out_hbm,
                 pix_v, bary_v, idx_v, rows_v, out_v, sem):
    info = plsc.get_sparse_core_info()
    wid = lax.axis_index("s") * info.num_cores + lax.axis_index("c")
    base = wid * (_PPW // _W)

    lane = lax.iota(jnp.int32, 16)
    cols_t = [jnp.full((16,), 3 * i + c, jnp.int32)
              for i in range(3) for c in range(3)]

    def chunk(g, _):
        nh0 = base + g * _RPC
        pltpu.sync_copy(pix_hbm.at[pl.ds(nh0, _RPC), :, 0], pix_v)
        for j in range(3):
            pltpu.sync_copy(bary_hbm.at[pl.ds(nh0, _RPC), j, :, 0],
                            bary_v.at[j])

        copies = []
        for j in range(_NG):
            def mkidx(i, _, j=j):
                ii = j * _GPR + i
                f = pix_v[ii // 32, (ii % 32) // 8, pl.ds((ii % 8) * 16, 16)]
                idx_v[j, pl.ds(i * 16, 16)] = jnp.maximum(f, 0)
                return _
            lax.fori_loop(0, _GPR, mkidx, None)
            copies.append(pltpu.async_copy(
                tex_hbm.at[idx_v.at[j]],
                rows_v.at[pl.ds(j * _GROW, _GROW)], sem))
        for cp in copies:
            cp.wait()

        def comp(i, _):
            r = i * 16 + lane
            d0, d1, d2 = i // 32, (i % 32) // 8, (i % 8) * 16
            f = pix_v[d0, d1, pl.ds(d2, 16)]
            valid = f >= 0
            w = [bary_v[j, d0, d1, pl.ds(d2, 16)] for j in range(3)]
            for c in range(3):
                acc = w[0] * plsc.load_gather(rows_v, [r, cols_t[c]])
                acc += w[1] * plsc.load_gather(rows_v, [r, cols_t[3 + c]])
                acc += w[2] * plsc.load_gather(rows_v, [r, cols_t[6 + c]])
                acc = jnp.where(valid, acc, jnp.zeros((16,), jnp.float32))
                plsc.store_scatter(out_v, [r * 3 + c], acc)
            return _
        lax.fori_loop(0, _B // 16, comp, None)

        pltpu.sync_copy(out_v, out_hbm.at[pl.ds(nh0 * _W * 3, _B * 3)])
        return _
    lax.fori_loop(0, _NCHUNK, chunk, None)


def kernel(pix_to_face, bary_coords, face_textures):
    pix = pix_to_face.astype(jnp.int32).reshape(_N, _H, _WT, 128, _K)
    pix = jnp.transpose(pix, (0, 1, 2, 4, 3)).reshape(_NH, _WT, _K, 128)
    bary = bary_coords.reshape(_N, _H, _WT, 128, _K, 3)
    bary = jnp.transpose(bary, (0, 1, 5, 2, 4, 3)).reshape(
        _NH, 3, _WT, _K, 128)
    tex = face_textures.reshape(-1, 9)
    tex = jnp.pad(tex, ((0, 0), (0, _D - 9)))

    mesh = plsc.VectorSubcoreMesh(core_axis_name="c", subcore_axis_name="s")
    run = functools.partial(
        pl.kernel,
        mesh=mesh,
        compiler_params=pltpu.CompilerParams(
            needs_layout_passes=False, use_tc_tiling_on_sc=False),
        out_type=jax.ShapeDtypeStruct((_P * 3,), jnp.float32),
        scratch_types=[
            pltpu.VMEM((_RPC, _WT, 128), jnp.int32),
            pltpu.VMEM((3, _RPC, _WT, 128), jnp.float32),
            pltpu.VMEM((_NG, _GROW), jnp.int32),
            pltpu.VMEM((_B, _D), jnp.float32),
            pltpu.VMEM((_B * 3,), jnp.float32),
            pltpu.SemaphoreType.DMA,
        ],
    )(_shader_body)
    out = run(pix, bary, tex)
    return out.reshape(_N, _H, _W, _C)


# speedup vs baseline: 227.0707x; 3.8132x over previous
import functools

import jax
import jax.numpy as jnp
from jax import lax
from jax.experimental import pallas as pl
from jax.experimental.pallas import tpu as pltpu
from jax.experimental.pallas import tpu_sc as plsc

_N, _H, _W, _K, _C = 8, 512, 512, 4, 3
_P = _N * _H * _W
_NH = _N * _H
_NW = 32
_PPW = _P // _NW
_B = 2048
_RPC = _B // _W
_NCHUNK = _PPW // _B
_GROW = 128
_NG = _B // _GROW
_GPR = _GROW // 16
_D = 16
_WT = _W // 128


def _shader_body(pix_hbm, bary_hbm, tex_hbm, out_hbm,
                 pix_v, bary_v, idx_v, rows_v, out_v, sem):
    info = plsc.get_sparse_core_info()
    wid = lax.axis_index("s") * info.num_cores + lax.axis_index("c")
    base = wid * (_PPW // _W)

    lane = lax.iota(jnp.int32, 16)
    cols_t = [jnp.full((16,), 3 * i + c, jnp.int32)
              for i in range(3) for c in range(3)]

    def chunk(g, _):
        nh0 = base + g * _RPC
        n = nh0 // _H
        ht = (nh0 - n * _H) // 8
        hl0 = nh0 % 8
        pltpu.sync_copy(pix_hbm.at[pl.ds(nh0, _RPC), :, 0], pix_v)
        pltpu.sync_copy(bary_hbm.at[pl.ds(nh0, _RPC), :, :, 0], bary_v)

        copies = []
        for j in range(_NG):
            def mkidx(i, _, j=j):
                ii = j * _GPR + i
                f = pix_v[ii // 32, (ii % 32) // 8, pl.ds((ii % 8) * 16, 16)]
                idx_v[j, pl.ds(i * 16, 16)] = jnp.maximum(f, 0)
                return _
            lax.fori_loop(0, _GPR, mkidx, None)
            copies.append(pltpu.async_copy(
                tex_hbm.at[idx_v.at[j]],
                rows_v.at[pl.ds(j * _GROW, _GROW)], sem))
        for cp in copies:
            cp.wait()

        def comp(i, _):
            r = i * 16 + lane
            d0, d1, d2 = i // 32, (i % 32) // 8, (i % 8) * 16
            f = pix_v[d0, d1, pl.ds(d2, 16)]
            valid = f >= 0
            w = [bary_v[d0, j, d1, pl.ds(d2, 16)] for j in range(3)]
            for c in range(3):
                acc = w[0] * plsc.load_gather(rows_v, [r, cols_t[c]])
                acc += w[1] * plsc.load_gather(rows_v, [r, cols_t[3 + c]])
                acc += w[2] * plsc.load_gather(rows_v, [r, cols_t[6 + c]])
                acc = jnp.where(valid, acc, jnp.zeros((16,), jnp.float32))
                out_v[c, d1, d0, pl.ds(d2, 16)] = acc
            return _
        lax.fori_loop(0, _B // 16, comp, None)

        pltpu.sync_copy(out_v, out_hbm.at[n, :, ht, :, pl.ds(hl0, _RPC), :])
        return _
    lax.fori_loop(0, _NCHUNK, chunk, None)


def kernel(pix_to_face, bary_coords, face_textures):
    pix = pix_to_face.astype(jnp.int32).reshape(_N, _H, _WT, 128, _K)
    pix = jnp.transpose(pix, (0, 1, 2, 4, 3)).reshape(_NH, _WT, _K, 128)
    bary = bary_coords.reshape(_N, _H, _WT, 128, _K, 3)
    bary = jnp.transpose(bary, (0, 1, 5, 2, 4, 3)).reshape(
        _NH, 3, _WT, _K, 128)
    tex = face_textures.reshape(-1, 9)
    tex = jnp.pad(tex, ((0, 0), (0, _D - 9)))

    mesh = plsc.VectorSubcoreMesh(core_axis_name="c", subcore_axis_name="s")
    run = functools.partial(
        pl.kernel,
        mesh=mesh,
        compiler_params=pltpu.CompilerParams(
            needs_layout_passes=False, use_tc_tiling_on_sc=False),
        out_type=jax.ShapeDtypeStruct((_N, _C, _H // 8, _WT, 8, 128),
                                      jnp.float32),
        scratch_types=[
            pltpu.VMEM((_RPC, _WT, 128), jnp.int32),
            pltpu.VMEM((_RPC, 3, _WT, 128), jnp.float32),
            pltpu.VMEM((_NG, _GROW), jnp.int32),
            pltpu.VMEM((_B, _D), jnp.float32),
            pltpu.VMEM((3, _WT, _RPC, 128), jnp.float32),
            pltpu.SemaphoreType.DMA,
        ],
    )(_shader_body)
    out = run(pix, bary, tex)
    out = jnp.transpose(out, (0, 1, 2, 4, 3, 5)).reshape(_N, _C, _H, _W)
    return jnp.transpose(out, (0, 2, 3, 1))


# speedup vs baseline: 243.3060x; 1.0715x over previous
import functools

import jax
import jax.numpy as jnp
from jax import lax
from jax.experimental import pallas as pl
from jax.experimental.pallas import tpu as pltpu
from jax.experimental.pallas import tpu_sc as plsc

_N, _H, _W, _K, _C = 8, 512, 512, 4, 3
_P = _N * _H * _W
_NH = _N * _H
_NW = 32
_PPW = _P // _NW
_B = 4096
_RPC = _B // _W
_NCHUNK = _PPW // _B
_GROW = 128
_NG = _B // _GROW
_GPR = _GROW // 16
_D = 16
_WT = _W // 128


def _shader_body(pix_hbm, bary_hbm, tex_hbm, out_hbm,
                 pix_v, bary_v, idx_v, rows_v, out_v, sem):
    info = plsc.get_sparse_core_info()
    wid = lax.axis_index("s") * info.num_cores + lax.axis_index("c")
    base = wid * (_PPW // _W)

    lane = lax.iota(jnp.int32, 16)
    cols_t = [jnp.full((16,), 3 * i + c, jnp.int32)
              for i in range(3) for c in range(3)]

    def chunk(g, _):
        nh0 = base + g * _RPC
        n = nh0 // _H
        ht = (nh0 - n * _H) // 8
        hl0 = nh0 % 8
        pltpu.sync_copy(pix_hbm.at[pl.ds(nh0, _RPC), :, 0], pix_v)
        pltpu.sync_copy(bary_hbm.at[pl.ds(nh0, _RPC), :, :, 0], bary_v)

        copies = []
        for j in range(_NG):
            def mkidx(i, _, j=j):
                ii = j * _GPR + i
                f = pix_v[ii // 32, (ii % 32) // 8, pl.ds((ii % 8) * 16, 16)]
                idx_v[j, pl.ds(i * 16, 16)] = jnp.maximum(f, 0)
                return _
            lax.fori_loop(0, _GPR, mkidx, None)
            copies.append(pltpu.async_copy(
                tex_hbm.at[idx_v.at[j]],
                rows_v.at[pl.ds(j * _GROW, _GROW)], sem))
        for cp in copies:
            cp.wait()

        def comp(i, _):
            r = i * 16 + lane
            d0, d1, d2 = i // 32, (i % 32) // 8, (i % 8) * 16
            f = pix_v[d0, d1, pl.ds(d2, 16)]
            valid = f >= 0
            w = [bary_v[d0, j, d1, pl.ds(d2, 16)] for j in range(3)]
            for c in range(3):
                acc = w[0] * plsc.load_gather(rows_v, [r, cols_t[c]])
                acc += w[1] * plsc.load_gather(rows_v, [r, cols_t[3 + c]])
                acc += w[2] * plsc.load_gather(rows_v, [r, cols_t[6 + c]])
                acc = jnp.where(valid, acc, jnp.zeros((16,), jnp.float32))
                out_v[c, d1, d0, pl.ds(d2, 16)] = acc
            return _
        lax.fori_loop(0, _B // 16, comp, None)

        pltpu.sync_copy(out_v, out_hbm.at[n, :, ht, :, pl.ds(hl0, _RPC), :])
        return _
    lax.fori_loop(0, _NCHUNK, chunk, None)


def kernel(pix_to_face, bary_coords, face_textures):
    pix = pix_to_face.astype(jnp.int32).reshape(_N, _H, _WT, 128, _K)
    pix = jnp.transpose(pix, (0, 1, 2, 4, 3)).reshape(_NH, _WT, _K, 128)
    bary = bary_coords.reshape(_N, _H, _WT, 128, _K, 3)
    bary = jnp.transpose(bary, (0, 1, 5, 2, 4, 3)).reshape(
        _NH, 3, _WT, _K, 128)
    tex = face_textures.reshape(-1, 9)
    tex = jnp.pad(tex, ((0, 0), (0, _D - 9)))

    mesh = plsc.VectorSubcoreMesh(core_axis_name="c", subcore_axis_name="s")
    run = functools.partial(
        pl.kernel,
        mesh=mesh,
        compiler_params=pltpu.CompilerParams(
            needs_layout_passes=False, use_tc_tiling_on_sc=False),
        out_type=jax.ShapeDtypeStruct((_N, _C, _H // 8, _WT, 8, 128),
                                      jnp.float32),
        scratch_types=[
            pltpu.VMEM((_RPC, _WT, 128), jnp.int32),
            pltpu.VMEM((_RPC, 3, _WT, 128), jnp.float32),
            pltpu.VMEM((_NG, _GROW), jnp.int32),
            pltpu.VMEM((_B, _D), jnp.float32),
            pltpu.VMEM((3, _WT, _RPC, 128), jnp.float32),
            pltpu.SemaphoreType.DMA,
        ],
    )(_shader_body)
    out = run(pix, bary, tex)
    out = jnp.transpose(out, (0, 1, 2, 4, 3, 5)).reshape(_N, _C, _H, _W)
    return jnp.transpose(out, (0, 2, 3, 1))


# speedup vs baseline: 276.8400x; 1.1378x over previous
import functools

import jax
import jax.numpy as jnp
from jax import lax
from jax.experimental import pallas as pl
from jax.experimental.pallas import tpu as pltpu
from jax.experimental.pallas import tpu_sc as plsc

_N, _H, _W, _K, _C = 8, 512, 512, 4, 3
_P = _N * _H * _W
_NH = _N * _H
_NW = 32
_PPW = _P // _NW
_B = 4096
_RPC = _B // _W
_NCHUNK = _PPW // _B
_GROW = 128
_NG = _B // _GROW
_GPR = _GROW // 16
_D = 16
_WT = _W // 128


def _shader_body(pix_hbm, bary_hbm, tex_hbm, out_hbm,
                 pix_v, bary_v, idx_v, rows_v, out_v, sem):
    info = plsc.get_sparse_core_info()
    wid = lax.axis_index("s") * info.num_cores + lax.axis_index("c")
    base = wid * (_PPW // _W)

    lane = lax.iota(jnp.int32, 16)
    cols_t = [jnp.full((16,), 3 * i + c, jnp.int32)
              for i in range(3) for c in range(3)]

    def chunk(g, _):
        nh0 = base + g * _RPC
        n = nh0 // _H
        ht = (nh0 - n * _H) // 8
        hl0 = nh0 % 8
        pltpu.sync_copy(pix_hbm.at[pl.ds(nh0, _RPC), :, 0], pix_v)
        pltpu.sync_copy(bary_hbm.at[pl.ds(nh0, _RPC), :, :, 0], bary_v)

        copies = []
        for j in range(_NG):
            def mkidx(i, _, j=j):
                ii = j * _GPR + i
                f = pix_v[ii // 32, (ii % 32) // 8, pl.ds((ii % 8) * 16, 16)]
                idx_v[j, pl.ds(i * 16, 16)] = jnp.maximum(f, 0)
                return _
            lax.fori_loop(0, _GPR, mkidx, None)
            copies.append(pltpu.async_copy(
                tex_hbm.at[idx_v.at[j]],
                rows_v.at[pl.ds(j * _GROW, _GROW)], sem))

        def comp(i, _):
            r = i * 16 + lane
            d0, d1, d2 = i // 32, (i % 32) // 8, (i % 8) * 16
            f = pix_v[d0, d1, pl.ds(d2, 16)]
            valid = f >= 0
            w = [bary_v[d0, j, d1, pl.ds(d2, 16)] for j in range(3)]
            for c in range(3):
                acc = w[0] * plsc.load_gather(rows_v, [r, cols_t[c]])
                acc += w[1] * plsc.load_gather(rows_v, [r, cols_t[3 + c]])
                acc += w[2] * plsc.load_gather(rows_v, [r, cols_t[6 + c]])
                acc = jnp.where(valid, acc, jnp.zeros((16,), jnp.float32))
                out_v[c, d1, d0, pl.ds(d2, 16)] = acc
            return _
        for j in range(_NG):
            copies[j].wait()
            lax.fori_loop(j * _GPR, (j + 1) * _GPR, comp, None)

        pltpu.sync_copy(out_v, out_hbm.at[n, :, ht, :, pl.ds(hl0, _RPC), :])
        return _
    lax.fori_loop(0, _NCHUNK, chunk, None)


def kernel(pix_to_face, bary_coords, face_textures):
    pix = pix_to_face.astype(jnp.int32).reshape(_N, _H, _WT, 128, _K)
    pix = jnp.transpose(pix, (0, 1, 2, 4, 3)).reshape(_NH, _WT, _K, 128)
    bary = bary_coords.reshape(_N, _H, _WT, 128, _K, 3)
    bary = jnp.transpose(bary, (0, 1, 5, 2, 4, 3)).reshape(
        _NH, 3, _WT, _K, 128)
    tex = face_textures.reshape(-1, 9)
    tex = jnp.pad(tex, ((0, 0), (0, _D - 9)))

    mesh = plsc.VectorSubcoreMesh(core_axis_name="c", subcore_axis_name="s")
    run = functools.partial(
        pl.kernel,
        mesh=mesh,
        compiler_params=pltpu.CompilerParams(
            needs_layout_passes=False, use_tc_tiling_on_sc=False),
        out_type=jax.ShapeDtypeStruct((_N, _C, _H // 8, _WT, 8, 128),
                                      jnp.float32),
        scratch_types=[
            pltpu.VMEM((_RPC, _WT, 128), jnp.int32),
            pltpu.VMEM((_RPC, 3, _WT, 128), jnp.float32),
            pltpu.VMEM((_NG, _GROW), jnp.int32),
            pltpu.VMEM((_B, _D), jnp.float32),
            pltpu.VMEM((3, _WT, _RPC, 128), jnp.float32),
            pltpu.SemaphoreType.DMA,
        ],
    )(_shader_body)
    out = run(pix, bary, tex)
    out = jnp.transpose(out, (0, 1, 2, 4, 3, 5)).reshape(_N, _C, _H, _W)
    return jnp.transpose(out, (0, 2, 3, 1))
